# Initial kernel scaffold; baseline (speedup 1.0000x reference)
import jax
import jax.numpy as jnp
from jax import lax
from jax.experimental import pallas as pl
from jax.experimental.pallas import tpu as pltpu
from jax.experimental.pallas import tpu_sc as plsc

B, N, M, C, NS = 4, 8192, 2048, 96, 32
L = 16
NW = 32
TPB = NW // B
MCHUNK = M // TPB
CAP = 64
NSTEPS = N // L
NGRP = MCHUNK * NS // L


def _cmp_ex(ka, va, kb, vb):
    m = ka <= kb
    lo_k = jnp.where(m, ka, kb)
    lo_v = jnp.where(m, va, vb)
    hi_k = jnp.where(m, kb, ka)
    hi_v = jnp.where(m, vb, va)
    return lo_k, lo_v, hi_k, hi_v


def _merge32(ka, va, kb, vb):
    rkb = jnp.flip(kb, 0)
    rvb = jnp.flip(vb, 0)
    lk, lv, hk, hv = _cmp_ex(ka, va, rkb, rvb)
    lk, lv = plsc.sort_key_val(lk, lv)
    hk, hv = plsc.sort_key_val(hk, hv)
    return lk, lv, hk, hv


def _low32(sk, sv):
    a0k, a0v, a1k, a1v = _merge32(sk[0], sv[0], sk[1], sv[1])
    b0k, b0v, b1k, b1v = _merge32(sk[2], sv[2], sk[3], sv[3])
    rb0k = jnp.flip(b1k, 0)
    rb0v = jnp.flip(b1v, 0)
    rb1k = jnp.flip(b0k, 0)
    rb1v = jnp.flip(b0v, 0)
    l0k, l0v, _, _ = _cmp_ex(a0k, a0v, rb0k, rb0v)
    l1k, l1v, _, _ = _cmp_ex(a1k, a1v, rb1k, rb1v)
    n0k, n0v, n1k, n1v = _cmp_ex(l0k, l0v, l1k, l1v)
    n0k, n0v = plsc.sort_key_val(n0k, n0v)
    n1k, n1v = plsc.sort_key_val(n1k, n1v)
    return n0k, n0v, n1k, n1v


def _body(xyzT, new_xyz, features, nf, gx,
          pts, qv, cd, ci, idxb, tab, ob, ob2, cnt_ref, t_ref):
    wid = lax.axis_index("s") * 2 + lax.axis_index("c")
    b = wid // TPB
    mlo = (wid % TPB) * MCHUNK

    pltpu.sync_copy(xyzT.at[b], pts)
    pltpu.sync_copy(new_xyz.at[b, pl.ds(mlo, MCHUNK)], qv)

    iota = lax.iota(jnp.int32, L)
    inf16 = jnp.full((L,), jnp.inf, jnp.float32)
    zero16 = jnp.zeros((L,), jnp.int32)

    def _select32():
        sk, sv = [], []
        for j in range(CAP // L):
            a, bvals = plsc.sort_key_val(cd[pl.ds(j * L, L)],
                                         ci[pl.ds(j * L, L)])
            sk.append(a)
            sv.append(bvals)
        n0k, n0v, n1k, n1v = _low32(sk, sv)
        cd[pl.ds(0, L)] = n0k
        cd[pl.ds(L, L)] = n1k
        ci[pl.ds(0, L)] = n0v
        ci[pl.ds(L, L)] = n1v
        for j in range(2, CAP // L):
            cd[pl.ds(j * L, L)] = inf16
        cnt_ref[0] = 2 * L
        t_ref[0] = jnp.max(n1k)

    def qbody(q, carry):
        for j in range(CAP // L):
            cd[pl.ds(j * L, L)] = inf16
        cnt_ref[0] = 0
        t_ref[0] = jnp.array(jnp.inf, jnp.float32)
        qidx = jnp.full((L,), q, jnp.int32)
        qxv = plsc.load_gather(qv, [qidx, zero16])
        qyv = plsc.load_gather(qv, [qidx, zero16 + 1])
        qzv = plsc.load_gather(qv, [qidx, zero16 + 2])

        def sbody(s, c):
            base = s * L
            px = pts[0, pl.ds(base, L)]
            py = pts[1, pl.ds(base, L)]
            pz = pts[2, pl.ds(base, L)]
            dx = px - qxv
            dy = py - qyv
            dz = pz - qzv
            d2 = dx * dx + dy * dy + dz * dz
            msk = d2 < t_ref[0]

            @pl.when(jnp.any(msk))
            def _():
                @pl.when(cnt_ref[0] > CAP - L)
                def _():
                    _select32()
                cnt = cnt_ref[0]
                iv = iota + base
                plsc.store_compressed(cd.at[pl.ds(cnt, L)], d2, msk)
                plsc.store_compressed(ci.at[pl.ds(cnt, L)], iv, msk)
                cnt_ref[0] = cnt + jnp.sum(msk.astype(jnp.int32))

            return c

        lax.fori_loop(0, NSTEPS, sbody, 0)
        _select32()
        idxb[pl.ds(q * NS, L)] = ci[pl.ds(0, L)]
        idxb[pl.ds(q * NS + L, L)] = ci[pl.ds(L, L)]
        return carry

    lax.fori_loop(0, MCHUNK, qbody, 0)

    for dch in range(3):
        didx = jnp.full((L,), dch, jnp.int32)

        def gbody(g, c, didx=didx):
            iv = idxb[pl.ds(g * L, L)]
            vals = plsc.load_gather(pts, [didx, iv])
            ob[pl.ds(g * L, L)] = vals
            qc = plsc.load_gather(qv, [jnp.full((L,), g // 2, jnp.int32), didx])
            ob2[pl.ds(g * L, L)] = vals - qc
            return c

        lax.fori_loop(0, NGRP, gbody, 0)
        pltpu.sync_copy(ob, gx.at[b, dch, pl.ds(mlo * NS, MCHUNK * NS)])
        pltpu.sync_copy(ob2, nf.at[b, dch, pl.ds(mlo * NS, MCHUNK * NS)])

    def cbody(ch, carry):
        pltpu.sync_copy(features.at[b, ch], tab)

        def gbody(g, c):
            iv = idxb[pl.ds(g * L, L)]
            ob[pl.ds(g * L, L)] = plsc.load_gather(tab, [iv])
            return c

        lax.fori_loop(0, NGRP, gbody, 0)
        pltpu.sync_copy(ob, nf.at[b, 3 + ch, pl.ds(mlo * NS, MCHUNK * NS)])
        return carry

    lax.fori_loop(0, C, cbody, 0)


def kernel(xyz, new_xyz, features):
    xyzT = jnp.transpose(xyz, (0, 2, 1))
    mesh = plsc.VectorSubcoreMesh(core_axis_name="c", subcore_axis_name="s",
                                  num_cores=2, num_subcores=16)
    nf, gx = pl.kernel(
        _body,
        out_type=[
            jax.ShapeDtypeStruct((B, 3 + C, M * NS), jnp.float32),
            jax.ShapeDtypeStruct((B, 3, M * NS), jnp.float32),
        ],
        mesh=mesh,
        scratch_types=[
            pltpu.VMEM((3, N), jnp.float32),
            pltpu.VMEM((MCHUNK, 3), jnp.float32),
            pltpu.VMEM((CAP,), jnp.float32),
            pltpu.VMEM((CAP,), jnp.int32),
            pltpu.VMEM((MCHUNK * NS,), jnp.int32),
            pltpu.VMEM((N,), jnp.float32),
            pltpu.VMEM((MCHUNK * NS,), jnp.float32),
            pltpu.VMEM((MCHUNK * NS,), jnp.float32),
            pltpu.SMEM((1,), jnp.int32),
            pltpu.SMEM((1,), jnp.float32),
        ],
    )(xyzT, new_xyz, features)
    return (nf.reshape(B, 3 + C, M, NS), gx.reshape(B, 3, M, NS))


# speedup vs baseline: 4.5614x; 4.5614x over previous
import jax
import jax.numpy as jnp
from jax import lax
from jax.experimental import pallas as pl
from jax.experimental.pallas import tpu as pltpu
from jax.experimental.pallas import tpu_sc as plsc

B, N, M, C, NS = 4, 8192, 2048, 96, 32
L = 16
NW = 32
TPB = NW // B
MCHUNK = M // TPB
CAP = 64
NSTEPS = N // L
NGRP = MCHUNK * NS // L
CH = 3 + C


def _cmp_ex(ka, va, kb, vb):
    m = ka <= kb
    lo_k = jnp.where(m, ka, kb)
    lo_v = jnp.where(m, va, vb)
    hi_k = jnp.where(m, kb, ka)
    hi_v = jnp.where(m, vb, va)
    return lo_k, lo_v, hi_k, hi_v


def _sort16(k, v):
    r = plsc.sort_key_val(k, v)
    return r[0], r[1]


def _merge32(ka, va, kb, vb):
    rkb = jnp.flip(kb, 0)
    rvb = jnp.flip(vb, 0)
    lk, lv, hk, hv = _cmp_ex(ka, va, rkb, rvb)
    lk, lv = _sort16(lk, lv)
    hk, hv = _sort16(hk, hv)
    return lk, lv, hk, hv


def _low32(sk, sv):
    a0k, a0v, a1k, a1v = _merge32(sk[0], sv[0], sk[1], sv[1])
    b0k, b0v, b1k, b1v = _merge32(sk[2], sv[2], sk[3], sv[3])
    rb0k = jnp.flip(b1k, 0)
    rb0v = jnp.flip(b1v, 0)
    rb1k = jnp.flip(b0k, 0)
    rb1v = jnp.flip(b0v, 0)
    l0k, l0v, _, _ = _cmp_ex(a0k, a0v, rb0k, rb0v)
    l1k, l1v, _, _ = _cmp_ex(a1k, a1v, rb1k, rb1v)
    n0k, n0v, n1k, n1v = _cmp_ex(l0k, l0v, l1k, l1v)
    n0k, n0v = _sort16(n0k, n0v)
    n1k, n1v = _sort16(n1k, n1v)
    return n0k, n0v, n1k, n1v


def _rne_bf16(x):
    u = plsc.bitcast(x, jnp.int32)
    r = u + 0x7FFF + ((u >> 16) & 1)
    r = r & jnp.int32(-65536)
    return plsc.bitcast(r, jnp.float32)


def _body(xyzT, new_xyz, features, nf, gx,
          pts, pb, x2v, qv, cd, ci, idxb, tab, ob, ob2, cnt_ref, t_ref):
    wid = lax.axis_index("s") * 2 + lax.axis_index("c")
    b = wid // TPB
    mlo = (wid % TPB) * MCHUNK

    pltpu.sync_copy(xyzT.at[pl.ds(b * 3 * N, 3 * N)], pts)
    pltpu.sync_copy(new_xyz.at[pl.ds((b * M + mlo) * 3, MCHUNK * 3)], qv)

    iota = lax.iota(jnp.int32, L)
    inf16 = jnp.full((L,), jnp.inf, jnp.float32)

    def pbody(s, c):
        base = s * L
        px = pts[pl.ds(base, L)]
        py = pts[pl.ds(N + base, L)]
        pz = pts[pl.ds(2 * N + base, L)]
        pb[pl.ds(base, L)] = _rne_bf16(px)
        pb[pl.ds(N + base, L)] = _rne_bf16(py)
        pb[pl.ds(2 * N + base, L)] = _rne_bf16(pz)
        x2v[pl.ds(base, L)] = (px * px + py * py) + pz * pz
        return c

    lax.fori_loop(0, NSTEPS, pbody, 0)

    def _select32():
        sk, sv = [], []
        for j in range(CAP // L):
            a, bvals = _sort16(cd[pl.ds(j * L, L)], ci[pl.ds(j * L, L)])
            sk.append(a)
            sv.append(bvals)
        n0k, n0v, n1k, n1v = _low32(sk, sv)
        cd[pl.ds(0, L)] = n0k
        cd[pl.ds(L, L)] = n1k
        ci[pl.ds(0, L)] = n0v
        ci[pl.ds(L, L)] = n1v
        for j in range(2, CAP // L):
            cd[pl.ds(j * L, L)] = inf16
        cnt_ref[0] = 2 * L
        t_ref[0] = jnp.max(n1k)

    def qbody(q, carry):
        for j in range(CAP // L):
            cd[pl.ds(j * L, L)] = inf16
        cnt_ref[0] = 0
        t_ref[0] = jnp.array(jnp.inf, jnp.float32)
        qxv = plsc.load_gather(qv, [jnp.full((L,), 3 * q, jnp.int32)])
        qyv = plsc.load_gather(qv, [jnp.full((L,), 3 * q + 1, jnp.int32)])
        qzv = plsc.load_gather(qv, [jnp.full((L,), 3 * q + 2, jnp.int32)])
        q2v = (qxv * qxv + qyv * qyv) + qzv * qzv
        qxb = _rne_bf16(qxv)
        qyb = _rne_bf16(qyv)
        qzb = _rne_bf16(qzv)

        def sbody(s, c):
            base = s * L
            px = pb[pl.ds(base, L)]
            py = pb[pl.ds(N + base, L)]
            pz = pb[pl.ds(2 * N + base, L)]
            x2 = x2v[pl.ds(base, L)]
            inner = px * qxb + py * qyb + pz * qzb
            d2 = (q2v - (inner + inner)) + x2
            msk = d2 < t_ref[0]

            @pl.when(jnp.any(msk))
            def _():
                @pl.when(cnt_ref[0] > CAP - L)
                def _():
                    _select32()
                cnt = cnt_ref[0]
                iv = iota + base
                plsc.store_compressed(cd.at[pl.ds(cnt, L)], d2, mask=msk)
                plsc.store_compressed(ci.at[pl.ds(cnt, L)], iv, mask=msk)
                cnt_ref[0] = cnt + jnp.sum(msk.astype(jnp.int32))

            return c

        lax.fori_loop(0, NSTEPS, sbody, 0)
        _select32()
        idxb[pl.ds(q * NS, L)] = ci[pl.ds(0, L)]
        idxb[pl.ds(q * NS + L, L)] = ci[pl.ds(L, L)]
        return carry

    lax.fori_loop(0, MCHUNK, qbody, 0)

    for dch in range(3):
        def gbody(g, c, dch=dch):
            iv = idxb[pl.ds(g * L, L)]
            vals = plsc.load_gather(pts, [iv + dch * N])
            ob[pl.ds(g * L, L)] = vals
            qc = plsc.load_gather(
                qv, [jnp.full((L,), (g // 2) * 3 + dch, jnp.int32)])
            ob2[pl.ds(g * L, L)] = vals - qc
            return c

        lax.fori_loop(0, NGRP, gbody, 0)
        pltpu.sync_copy(
            ob, gx.at[pl.ds(((b * 3 + dch) * M + mlo) * NS, MCHUNK * NS)])
        pltpu.sync_copy(
            ob2, nf.at[pl.ds(((b * CH + dch) * M + mlo) * NS, MCHUNK * NS)])

    def cbody(ch, carry):
        pltpu.sync_copy(features.at[pl.ds((b * C + ch) * N, N)], tab)

        def gbody(g, c):
            iv = idxb[pl.ds(g * L, L)]
            ob[pl.ds(g * L, L)] = plsc.load_gather(tab, [iv])
            return c

        lax.fori_loop(0, NGRP, gbody, 0)
        pltpu.sync_copy(
            ob, nf.at[pl.ds(((b * CH + 3 + ch) * M + mlo) * NS, MCHUNK * NS)])
        return carry

    lax.fori_loop(0, C, cbody, 0)


def kernel(xyz, new_xyz, features):
    xyzT = jnp.transpose(xyz, (0, 2, 1)).reshape(-1)
    mesh = plsc.VectorSubcoreMesh(core_axis_name="c", subcore_axis_name="s",
                                  num_cores=2, num_subcores=16)
    nf, gx = pl.kernel(
        _body,
        out_type=[
            jax.ShapeDtypeStruct((B * CH * M * NS,), jnp.float32),
            jax.ShapeDtypeStruct((B * 3 * M * NS,), jnp.float32),
        ],
        mesh=mesh,
        compiler_params=pltpu.CompilerParams(needs_layout_passes=False),
        scratch_types=[
            pltpu.VMEM((3 * N,), jnp.float32),
            pltpu.VMEM((3 * N,), jnp.float32),
            pltpu.VMEM((N,), jnp.float32),
            pltpu.VMEM((MCHUNK * 3,), jnp.float32),
            pltpu.VMEM((CAP,), jnp.float32),
            pltpu.VMEM((CAP,), jnp.int32),
            pltpu.VMEM((MCHUNK * NS,), jnp.int32),
            pltpu.VMEM((N,), jnp.float32),
            pltpu.VMEM((MCHUNK * NS,), jnp.float32),
            pltpu.VMEM((MCHUNK * NS,), jnp.float32),
            pltpu.SMEM((1,), jnp.int32),
            pltpu.SMEM((1,), jnp.float32),
        ],
    )(xyzT, new_xyz.reshape(-1), features.reshape(-1))
    return (nf.reshape(B, CH, M, NS), gx.reshape(B, 3, M, NS))


# speedup vs baseline: 7.8455x; 1.7200x over previous
import jax
import jax.numpy as jnp
from jax import lax
from jax.experimental import pallas as pl
from jax.experimental.pallas import tpu as pltpu
from jax.experimental.pallas import tpu_sc as plsc

B, N, M, C, NS = 4, 8192, 2048, 96, 32
L = 16
NW = 32
TPB = NW // B
MCHUNK = M // TPB
NSTEPS = N // L
NGRP = MCHUNK * NS // L
CH = 3 + C
WD = 16


_GDN = lax.GatherDimensionNumbers(offset_dims=(), collapsed_slice_dims=(0,),
                                  start_index_map=(0,))


def _take16(x, idxcol):
    return lax.gather(x, idxcol, _GDN, (1,),
                      mode=lax.GatherScatterMode.PROMISE_IN_BOUNDS)


def _cmp_ex(ka, va, kb, vb):
    m = (ka < kb) | ((ka == kb) & (va <= vb))
    lo_k = jnp.where(m, ka, kb)
    lo_v = jnp.where(m, va, vb)
    hi_k = jnp.where(m, kb, ka)
    hi_v = jnp.where(m, vb, va)
    return lo_k, lo_v, hi_k, hi_v


def _sort16(k, v):
    r = plsc.sort_key_val(k, v)
    return r[0], r[1]


def _bsort16(k, v, net):
    for idxcol, low in net:
        pk = _take16(k, idxcol)
        pv = _take16(v, idxcol)
        m = (k < pk) | ((k == pk) & (v <= pv))
        kmin = jnp.where(m, k, pk)
        kmax = jnp.where(m, pk, k)
        vmin = jnp.where(m, v, pv)
        vmax = jnp.where(m, pv, v)
        k = jnp.where(low, kmin, kmax)
        v = jnp.where(low, vmin, vmax)
    return k, v


def _merge32(ka, va, kb, vb, net):
    rkb = jnp.flip(kb, 0)
    rvb = jnp.flip(vb, 0)
    lk, lv, hk, hv = _cmp_ex(ka, va, rkb, rvb)
    lk, lv = _bsort16(lk, lv, net)
    hk, hv = _bsort16(hk, hv, net)
    return lk, lv, hk, hv


def _low32pair(ak0, av0, ak1, av1, bk0, bv0, bk1, bv1, net):
    rbk0 = jnp.flip(bk1, 0)
    rbv0 = jnp.flip(bv1, 0)
    rbk1 = jnp.flip(bk0, 0)
    rbv1 = jnp.flip(bv0, 0)
    l0k, l0v, _, _ = _cmp_ex(ak0, av0, rbk0, rbv0)
    l1k, l1v, _, _ = _cmp_ex(ak1, av1, rbk1, rbv1)
    n0k, n0v, n1k, n1v = _cmp_ex(l0k, l0v, l1k, l1v)
    n0k, n0v = _bsort16(n0k, n0v, net)
    n1k, n1v = _bsort16(n1k, n1v, net)
    return n0k, n0v, n1k, n1v


def _ins3(m1, m2, m3, x):
    a = jnp.maximum(m1, x)
    m1 = jnp.minimum(m1, x)
    bv = jnp.maximum(m2, a)
    m2 = jnp.minimum(m2, a)
    m3 = jnp.minimum(m3, bv)
    return m1, m2, m3


def _rne_bf16(x):
    u = plsc.bitcast(x, jnp.int32)
    r = u + 0x7FFF + ((u >> 16) & 1)
    r = r & jnp.int32(-65536)
    return plsc.bitcast(r, jnp.float32)


def _body(xyzT, new_xyz, features, nf, gx,
          pts, pb, x2v, dstore, qv, wd, wi, idxb, tab, ob, ob2):
    wid = lax.axis_index("s") * 2 + lax.axis_index("c")
    b = wid // TPB
    mlo = (wid % TPB) * MCHUNK

    pltpu.sync_copy(xyzT.at[pl.ds(b * 3 * N, 3 * N)], pts)
    pltpu.sync_copy(new_xyz.at[pl.ds((b * M + mlo) * 3, MCHUNK * 3)], qv)

    iota = lax.iota(jnp.int32, L)
    inf16 = jnp.full((L,), jnp.inf, jnp.float32)
    zero16i = jnp.zeros((L,), jnp.int32)
    laneoff = iota * WD
    net = tuple((jnp.reshape(iota ^ s, (L, 1)), (iota & s) == 0)
                for s in (8, 4, 2, 1))

    def pbody(s, c):
        base = s * L
        px = pts[pl.ds(base, L)]
        py = pts[pl.ds(N + base, L)]
        pz = pts[pl.ds(2 * N + base, L)]
        pb[pl.ds(base, L)] = _rne_bf16(px)
        pb[pl.ds(N + base, L)] = _rne_bf16(py)
        pb[pl.ds(2 * N + base, L)] = _rne_bf16(pz)
        x2v[pl.ds(base, L)] = (px * px + py * py) + pz * pz
        return c

    lax.fori_loop(0, NSTEPS, pbody, 0)

    def qbody(q, carry):
        qxv = plsc.load_gather(qv, [jnp.full((L,), 3 * q, jnp.int32)])
        qyv = plsc.load_gather(qv, [jnp.full((L,), 3 * q + 1, jnp.int32)])
        qzv = plsc.load_gather(qv, [jnp.full((L,), 3 * q + 2, jnp.int32)])
        q2v = (qxv * qxv + qyv * qyv) + qzv * qzv
        qxb = _rne_bf16(qxv)
        qyb = _rne_bf16(qyv)
        qzb = _rne_bf16(qzv)

        def abody(s2, carry):
            m1a, m2a, m3a, m1b, m2b, m3b = carry
            base = s2 * (2 * L)
            px = pb[pl.ds(base, L)]
            py = pb[pl.ds(N + base, L)]
            pz = pb[pl.ds(2 * N + base, L)]
            x2 = x2v[pl.ds(base, L)]
            inner = px * qxb + py * qyb + pz * qzb
            d2 = (q2v - (inner + inner)) + x2
            dstore[pl.ds(base, L)] = d2
            m1a, m2a, m3a = _ins3(m1a, m2a, m3a, d2)
            base = base + L
            px = pb[pl.ds(base, L)]
            py = pb[pl.ds(N + base, L)]
            pz = pb[pl.ds(2 * N + base, L)]
            x2 = x2v[pl.ds(base, L)]
            inner = px * qxb + py * qyb + pz * qzb
            d2 = (q2v - (inner + inner)) + x2
            dstore[pl.ds(base, L)] = d2
            m1b, m2b, m3b = _ins3(m1b, m2b, m3b, d2)
            return m1a, m2a, m3a, m1b, m2b, m3b

        m1, m2, m3, m1b, m2b, m3b = lax.fori_loop(
            0, NSTEPS // 2, abody, (inf16,) * 6)
        m1, m2, m3 = _ins3(m1, m2, m3, m1b)
        m1, m2, m3 = _ins3(m1, m2, m3, m2b)
        m1, m2, m3 = _ins3(m1, m2, m3, m3b)
        srt = jnp.sort(m3)
        t0v = _take16(srt, jnp.full((L, 1), 10, jnp.int32))

        for r in range(WD):
            wd[pl.ds(r * L, L)] = inf16

        def bbody(s, cntv):
            base = s * L
            d2 = dstore[pl.ds(base, L)]
            msk = d2 <= t0v
            pos = laneoff + jnp.minimum(cntv, WD - 1)
            plsc.store_scatter(wd, [pos], d2, mask=msk)
            plsc.store_scatter(wi, [pos], iota + base, mask=msk)
            return cntv + msk.astype(jnp.int32)

        cntv = lax.fori_loop(0, NSTEPS, bbody, zero16i)
        cntmax = jnp.max(cntv)

        @pl.when(cntmax <= WD)
        def _():
            quads = []
            for qd in range(4):
                runs = []
                for r in range(4):
                    rr = qd * 4 + r
                    runs.append(_sort16(wd[pl.ds(rr * L, L)],
                                        wi[pl.ds(rr * L, L)]))
                A = _merge32(*runs[0], *runs[1], net)
                Bm = _merge32(*runs[2], *runs[3], net)
                quads.append(_low32pair(*A, *Bm, net))
            AB = _low32pair(*quads[0], *quads[1], net)
            CD = _low32pair(*quads[2], *quads[3], net)
            n0k, n0v, n1k, n1v = _low32pair(*AB, *CD, net)
            idxb[pl.ds(q * NS, L)] = n0v
            idxb[pl.ds(q * NS + L, L)] = n1v

        @pl.when(cntmax > WD)
        def _():
            def fb(s, carry):
                k0, v0, k1, v1 = carry
                base = s * L
                dk = dstore[pl.ds(base, L)]
                dv = iota + base
                dk, dv = _sort16(dk, dv)
                rdk = jnp.flip(dk, 0)
                rdv = jnp.flip(dv, 0)
                lk, lv, _, _ = _cmp_ex(k1, v1, rdk, rdv)
                lk, lv = _bsort16(lk, lv, net)
                return _merge32(k0, v0, lk, lv, net)

            k0, v0, k1, v1 = lax.fori_loop(
                0, NSTEPS, fb, (inf16, zero16i, inf16, zero16i))
            idxb[pl.ds(q * NS, L)] = v0
            idxb[pl.ds(q * NS + L, L)] = v1

        return carry

    lax.fori_loop(0, MCHUNK, qbody, 0)

    for dch in range(3):
        def gbody(g, c, dch=dch):
            iv = idxb[pl.ds(g * L, L)]
            vals = plsc.load_gather(pts, [iv + dch * N])
            ob[pl.ds(g * L, L)] = vals
            qc = plsc.load_gather(
                qv, [jnp.full((L,), (g // 2) * 3 + dch, jnp.int32)])
            ob2[pl.ds(g * L, L)] = vals - qc
            return c

        lax.fori_loop(0, NGRP, gbody, 0)
        pltpu.sync_copy(
            ob, gx.at[pl.ds(((b * 3 + dch) * M + mlo) * NS, MCHUNK * NS)])
        pltpu.sync_copy(
            ob2, nf.at[pl.ds(((b * CH + dch) * M + mlo) * NS, MCHUNK * NS)])

    def cbody(ch, carry):
        pltpu.sync_copy(features.at[pl.ds((b * C + ch) * N, N)], tab)

        def gbody(g, c):
            iv = idxb[pl.ds(g * L, L)]
            ob[pl.ds(g * L, L)] = plsc.load_gather(tab, [iv])
            return c

        lax.fori_loop(0, NGRP, gbody, 0)
        pltpu.sync_copy(
            ob, nf.at[pl.ds(((b * CH + 3 + ch) * M + mlo) * NS, MCHUNK * NS)])
        return carry

    lax.fori_loop(0, C, cbody, 0)


def kernel(xyz, new_xyz, features):
    xyzT = jnp.transpose(xyz, (0, 2, 1)).reshape(-1)
    mesh = plsc.VectorSubcoreMesh(core_axis_name="c", subcore_axis_name="s",
                                  num_cores=2, num_subcores=16)
    nf, gx = pl.kernel(
        _body,
        out_type=[
            jax.ShapeDtypeStruct((B * CH * M * NS,), jnp.float32),
            jax.ShapeDtypeStruct((B * 3 * M * NS,), jnp.float32),
        ],
        mesh=mesh,
        compiler_params=pltpu.CompilerParams(needs_layout_passes=False),
        scratch_types=[
            pltpu.VMEM((3 * N,), jnp.float32),
            pltpu.VMEM((3 * N,), jnp.float32),
            pltpu.VMEM((N,), jnp.float32),
            pltpu.VMEM((N,), jnp.float32),
            pltpu.VMEM((MCHUNK * 3,), jnp.float32),
            pltpu.VMEM((WD * L,), jnp.float32),
            pltpu.VMEM((WD * L,), jnp.int32),
            pltpu.VMEM((MCHUNK * NS,), jnp.int32),
            pltpu.VMEM((N,), jnp.float32),
            pltpu.VMEM((MCHUNK * NS,), jnp.float32),
            pltpu.VMEM((MCHUNK * NS,), jnp.float32),
        ],
    )(xyzT, new_xyz.reshape(-1), features.reshape(-1))
    return (nf.reshape(B, CH, M, NS), gx.reshape(B, 3, M, NS))


# speedup vs baseline: 11.6111x; 1.4800x over previous
import jax
import jax.numpy as jnp
from jax import lax
from jax.experimental import pallas as pl
from jax.experimental.pallas import tpu as pltpu
from jax.experimental.pallas import tpu_sc as plsc

B, N, M, C, NS = 4, 8192, 2048, 96, 32
L = 16
NW = 32
TPB = NW // B
MCHUNK = M // TPB
NSTEPS = N // L
NGRP = MCHUNK * NS // L
CH = 3 + C
WD = 16


_GDN = lax.GatherDimensionNumbers(offset_dims=(), collapsed_slice_dims=(0,),
                                  start_index_map=(0,))


def _take16(x, idxcol):
    return lax.gather(x, idxcol, _GDN, (1,),
                      mode=lax.GatherScatterMode.PROMISE_IN_BOUNDS)


def _cmp_ex(ka, va, kb, vb):
    m = (ka < kb) | ((ka == kb) & (va <= vb))
    lo_k = jnp.where(m, ka, kb)
    lo_v = jnp.where(m, va, vb)
    hi_k = jnp.where(m, kb, ka)
    hi_v = jnp.where(m, vb, va)
    return lo_k, lo_v, hi_k, hi_v


def _sort16(k, v):
    r = plsc.sort_key_val(k, v)
    return r[0], r[1]


def _bsort16(k, v, net):
    for idxcol, low in net:
        pk = _take16(k, idxcol)
        pv = _take16(v, idxcol)
        m = (k < pk) | ((k == pk) & (v <= pv))
        kmin = jnp.where(m, k, pk)
        kmax = jnp.where(m, pk, k)
        vmin = jnp.where(m, v, pv)
        vmax = jnp.where(m, pv, v)
        k = jnp.where(low, kmin, kmax)
        v = jnp.where(low, vmin, vmax)
    return k, v


def _merge32(ka, va, kb, vb, net):
    rkb = jnp.flip(kb, 0)
    rvb = jnp.flip(vb, 0)
    lk, lv, hk, hv = _cmp_ex(ka, va, rkb, rvb)
    lk, lv = _bsort16(lk, lv, net)
    hk, hv = _bsort16(hk, hv, net)
    return lk, lv, hk, hv


def _low32pair(ak0, av0, ak1, av1, bk0, bv0, bk1, bv1, net):
    rbk0 = jnp.flip(bk1, 0)
    rbv0 = jnp.flip(bv1, 0)
    rbk1 = jnp.flip(bk0, 0)
    rbv1 = jnp.flip(bv0, 0)
    l0k, l0v, _, _ = _cmp_ex(ak0, av0, rbk0, rbv0)
    l1k, l1v, _, _ = _cmp_ex(ak1, av1, rbk1, rbv1)
    n0k, n0v, n1k, n1v = _cmp_ex(l0k, l0v, l1k, l1v)
    n0k, n0v = _bsort16(n0k, n0v, net)
    n1k, n1v = _bsort16(n1k, n1v, net)
    return n0k, n0v, n1k, n1v


def _ins3(m1, m2, m3, x):
    a = jnp.maximum(m1, x)
    m1 = jnp.minimum(m1, x)
    bv = jnp.maximum(m2, a)
    m2 = jnp.minimum(m2, a)
    m3 = jnp.minimum(m3, bv)
    return m1, m2, m3


def _rne_bf16(x):
    u = plsc.bitcast(x, jnp.int32)
    r = u + 0x7FFF + ((u >> 16) & 1)
    r = r & jnp.int32(-65536)
    return plsc.bitcast(r, jnp.float32)


def _body(xyzT, new_xyz, features, nf, gx,
          pts, pb, x2v, dstore, qv, wd, wi, idxb, tab, ob, ob2):
    wid = lax.axis_index("s") * 2 + lax.axis_index("c")
    b = wid // TPB
    mlo = (wid % TPB) * MCHUNK

    pltpu.sync_copy(xyzT.at[pl.ds(b * 3 * N, 3 * N)], pts)
    pltpu.sync_copy(new_xyz.at[pl.ds((b * M + mlo) * 3, MCHUNK * 3)], qv)

    iota = lax.iota(jnp.int32, L)
    inf16 = jnp.full((L,), jnp.inf, jnp.float32)
    zero16i = jnp.zeros((L,), jnp.int32)
    laneoff = iota * WD
    net = tuple((jnp.reshape(iota ^ s, (L, 1)), (iota & s) == 0)
                for s in (8, 4, 2, 1))

    def pbody(s, c):
        base = s * L
        px = pts[pl.ds(base, L)]
        py = pts[pl.ds(N + base, L)]
        pz = pts[pl.ds(2 * N + base, L)]
        pb[pl.ds(base, L)] = _rne_bf16(px)
        pb[pl.ds(N + base, L)] = _rne_bf16(py)
        pb[pl.ds(2 * N + base, L)] = _rne_bf16(pz)
        x2v[pl.ds(base, L)] = (px * px + py * py) + pz * pz
        return c

    lax.fori_loop(0, NSTEPS, pbody, 0)

    def qbody(q, carry):
        qxv = plsc.load_gather(qv, [jnp.full((L,), 3 * q, jnp.int32)])
        qyv = plsc.load_gather(qv, [jnp.full((L,), 3 * q + 1, jnp.int32)])
        qzv = plsc.load_gather(qv, [jnp.full((L,), 3 * q + 2, jnp.int32)])
        q2v = (qxv * qxv + qyv * qyv) + qzv * qzv
        qxb = _rne_bf16(qxv)
        qyb = _rne_bf16(qyv)
        qzb = _rne_bf16(qzv)

        def abody(s4, carry):
            tris = list(carry)
            base0 = s4 * (4 * L)
            for u in range(4):
                base = base0 + u * L
                px = pb[pl.ds(base, L)]
                py = pb[pl.ds(N + base, L)]
                pz = pb[pl.ds(2 * N + base, L)]
                x2 = x2v[pl.ds(base, L)]
                inner = px * qxb + py * qyb + pz * qzb
                d2 = (q2v - (inner + inner)) + x2
                dstore[pl.ds(base, L)] = d2
                tris[3 * u], tris[3 * u + 1], tris[3 * u + 2] = _ins3(
                    tris[3 * u], tris[3 * u + 1], tris[3 * u + 2], d2)
            return tuple(tris)

        tris = lax.fori_loop(0, NSTEPS // 4, abody, (inf16,) * 12)
        m1, m2, m3 = tris[0], tris[1], tris[2]
        for u in range(1, 4):
            for x in (tris[3 * u], tris[3 * u + 1], tris[3 * u + 2]):
                m1, m2, m3 = _ins3(m1, m2, m3, x)
        srt = jnp.sort(m3)
        t0v = _take16(srt, jnp.full((L, 1), 10, jnp.int32))

        for r in range(WD):
            wd[pl.ds(r * L, L)] = inf16

        def bbody(s4, cntv):
            base0 = s4 * (4 * L)
            for u in range(4):
                base = base0 + u * L
                d2 = dstore[pl.ds(base, L)]
                msk = d2 <= t0v
                pos = laneoff + jnp.minimum(cntv, WD - 1)
                plsc.store_scatter(wd, [pos], d2, mask=msk)
                plsc.store_scatter(wi, [pos], iota + base, mask=msk)
                cntv = cntv + msk.astype(jnp.int32)
            return cntv

        cntv = lax.fori_loop(0, NSTEPS // 4, bbody, zero16i)
        cntmax = jnp.max(cntv)

        @pl.when(cntmax <= WD)
        def _():
            quads = []
            for qd in range(4):
                runs = []
                for r in range(4):
                    rr = qd * 4 + r
                    runs.append(_sort16(wd[pl.ds(rr * L, L)],
                                        wi[pl.ds(rr * L, L)]))
                A = _merge32(*runs[0], *runs[1], net)
                Bm = _merge32(*runs[2], *runs[3], net)
                quads.append(_low32pair(*A, *Bm, net))
            AB = _low32pair(*quads[0], *quads[1], net)
            CD = _low32pair(*quads[2], *quads[3], net)
            n0k, n0v, n1k, n1v = _low32pair(*AB, *CD, net)
            idxb[pl.ds(q * NS, L)] = n0v
            idxb[pl.ds(q * NS + L, L)] = n1v

        @pl.when(cntmax > WD)
        def _():
            def fb(s, carry):
                k0, v0, k1, v1 = carry
                base = s * L
                dk = dstore[pl.ds(base, L)]
                dv = iota + base
                dk, dv = _sort16(dk, dv)
                rdk = jnp.flip(dk, 0)
                rdv = jnp.flip(dv, 0)
                lk, lv, _, _ = _cmp_ex(k1, v1, rdk, rdv)
                lk, lv = _bsort16(lk, lv, net)
                return _merge32(k0, v0, lk, lv, net)

            k0, v0, k1, v1 = lax.fori_loop(
                0, NSTEPS, fb, (inf16, zero16i, inf16, zero16i))
            idxb[pl.ds(q * NS, L)] = v0
            idxb[pl.ds(q * NS + L, L)] = v1

        return carry

    lax.fori_loop(0, MCHUNK, qbody, 0)

    for dch in range(3):
        def gbody(g, c, dch=dch):
            iv = idxb[pl.ds(g * L, L)]
            vals = plsc.load_gather(pts, [iv + dch * N])
            ob[pl.ds(g * L, L)] = vals
            qc = plsc.load_gather(
                qv, [jnp.full((L,), (g // 2) * 3 + dch, jnp.int32)])
            ob2[pl.ds(g * L, L)] = vals - qc
            return c

        lax.fori_loop(0, NGRP, gbody, 0)
        pltpu.sync_copy(
            ob, gx.at[pl.ds(((b * 3 + dch) * M + mlo) * NS, MCHUNK * NS)])
        pltpu.sync_copy(
            ob2, nf.at[pl.ds(((b * CH + dch) * M + mlo) * NS, MCHUNK * NS)])

    def cbody(ch, carry):
        pltpu.sync_copy(features.at[pl.ds((b * C + ch) * N, N)], tab)

        def gbody(g, c):
            iv = idxb[pl.ds(g * L, L)]
            ob[pl.ds(g * L, L)] = plsc.load_gather(tab, [iv])
            return c

        lax.fori_loop(0, NGRP, gbody, 0)
        pltpu.sync_copy(
            ob, nf.at[pl.ds(((b * CH + 3 + ch) * M + mlo) * NS, MCHUNK * NS)])
        return carry

    lax.fori_loop(0, C, cbody, 0)


def kernel(xyz, new_xyz, features):
    xyzT = jnp.transpose(xyz, (0, 2, 1)).reshape(-1)
    mesh = plsc.VectorSubcoreMesh(core_axis_name="c", subcore_axis_name="s",
                                  num_cores=2, num_subcores=16)
    nf, gx = pl.kernel(
        _body,
        out_type=[
            jax.ShapeDtypeStruct((B * CH * M * NS,), jnp.float32),
            jax.ShapeDtypeStruct((B * 3 * M * NS,), jnp.float32),
        ],
        mesh=mesh,
        compiler_params=pltpu.CompilerParams(needs_layout_passes=False),
        scratch_types=[
            pltpu.VMEM((3 * N,), jnp.float32),
            pltpu.VMEM((3 * N,), jnp.float32),
            pltpu.VMEM((N,), jnp.float32),
            pltpu.VMEM((N,), jnp.float32),
            pltpu.VMEM((MCHUNK * 3,), jnp.float32),
            pltpu.VMEM((WD * L,), jnp.float32),
            pltpu.VMEM((WD * L,), jnp.int32),
            pltpu.VMEM((MCHUNK * NS,), jnp.int32),
            pltpu.VMEM((N,), jnp.float32),
            pltpu.VMEM((MCHUNK * NS,), jnp.float32),
            pltpu.VMEM((MCHUNK * NS,), jnp.float32),
        ],
    )(xyzT, new_xyz.reshape(-1), features.reshape(-1))
    return (nf.reshape(B, CH, M, NS), gx.reshape(B, 3, M, NS))


# speedup vs baseline: 12.6635x; 1.0906x over previous
import jax
import jax.numpy as jnp
from jax import lax
from jax.experimental import pallas as pl
from jax.experimental.pallas import tpu as pltpu
from jax.experimental.pallas import tpu_sc as plsc

B, N, M, C, NS = 4, 8192, 2048, 96, 32
L = 16
NW = 32
TPB = NW // B
MCHUNK = M // TPB
NSTEPS = N // L
NGRP = MCHUNK * NS // L
CH = 3 + C
WD = 16


_GDN = lax.GatherDimensionNumbers(offset_dims=(), collapsed_slice_dims=(0,),
                                  start_index_map=(0,))


def _take16(x, idxcol):
    return lax.gather(x, idxcol, _GDN, (1,),
                      mode=lax.GatherScatterMode.PROMISE_IN_BOUNDS)


def _cmp_ex(ka, va, kb, vb):
    m = (ka < kb) | ((ka == kb) & (va <= vb))
    lo_k = jnp.where(m, ka, kb)
    lo_v = jnp.where(m, va, vb)
    hi_k = jnp.where(m, kb, ka)
    hi_v = jnp.where(m, vb, va)
    return lo_k, lo_v, hi_k, hi_v


def _sort16(k, v):
    r = plsc.sort_key_val(k, v)
    return r[0], r[1]


def _bsort16(k, v, net):
    for idxcol, low in net:
        pk = _take16(k, idxcol)
        pv = _take16(v, idxcol)
        m = (k < pk) | ((k == pk) & (v <= pv))
        kmin = jnp.where(m, k, pk)
        kmax = jnp.where(m, pk, k)
        vmin = jnp.where(m, v, pv)
        vmax = jnp.where(m, pv, v)
        k = jnp.where(low, kmin, kmax)
        v = jnp.where(low, vmin, vmax)
    return k, v


def _merge32(ka, va, kb, vb, net):
    rkb = jnp.flip(kb, 0)
    rvb = jnp.flip(vb, 0)
    lk, lv, hk, hv = _cmp_ex(ka, va, rkb, rvb)
    lk, lv = _bsort16(lk, lv, net)
    hk, hv = _bsort16(hk, hv, net)
    return lk, lv, hk, hv


def _low32pair(ak0, av0, ak1, av1, bk0, bv0, bk1, bv1, net):
    rbk0 = jnp.flip(bk1, 0)
    rbv0 = jnp.flip(bv1, 0)
    rbk1 = jnp.flip(bk0, 0)
    rbv1 = jnp.flip(bv0, 0)
    l0k, l0v, _, _ = _cmp_ex(ak0, av0, rbk0, rbv0)
    l1k, l1v, _, _ = _cmp_ex(ak1, av1, rbk1, rbv1)
    n0k, n0v, n1k, n1v = _cmp_ex(l0k, l0v, l1k, l1v)
    n0k, n0v = _bsort16(n0k, n0v, net)
    n1k, n1v = _bsort16(n1k, n1v, net)
    return n0k, n0v, n1k, n1v


def _ins3(m1, m2, m3, x):
    a = jnp.maximum(m1, x)
    m1 = jnp.minimum(m1, x)
    bv = jnp.maximum(m2, a)
    m2 = jnp.minimum(m2, a)
    m3 = jnp.minimum(m3, bv)
    return m1, m2, m3


def _rne_bf16(x):
    u = plsc.bitcast(x, jnp.int32)
    r = u + 0x7FFF + ((u >> 16) & 1)
    r = r & jnp.int32(-65536)
    return plsc.bitcast(r, jnp.float32)


def _body(xyzT, new_xyz, features, nf, gx,
          pts, pb, x2v, dstore, qv, wd, wi, idxb, ob, ob2, sem0, sem1):
    wid = lax.axis_index("s") * 2 + lax.axis_index("c")
    b = wid // TPB
    mlo = (wid % TPB) * MCHUNK

    pltpu.sync_copy(xyzT.at[pl.ds(b * 3 * N, 3 * N)], pts)
    pltpu.sync_copy(new_xyz.at[pl.ds((b * M + mlo) * 3, MCHUNK * 3)], qv)

    iota = lax.iota(jnp.int32, L)
    inf16 = jnp.full((L,), jnp.inf, jnp.float32)
    zero16i = jnp.zeros((L,), jnp.int32)
    laneoff = iota * WD
    net = tuple((jnp.reshape(iota ^ s, (L, 1)), (iota & s) == 0)
                for s in (8, 4, 2, 1))

    def pbody(s, c):
        base = s * L
        px = pts[pl.ds(base, L)]
        py = pts[pl.ds(N + base, L)]
        pz = pts[pl.ds(2 * N + base, L)]
        pb[pl.ds(base, L)] = _rne_bf16(px)
        pb[pl.ds(N + base, L)] = _rne_bf16(py)
        pb[pl.ds(2 * N + base, L)] = _rne_bf16(pz)
        x2v[pl.ds(base, L)] = (px * px + py * py) + pz * pz
        return c

    lax.fori_loop(0, NSTEPS, pbody, 0)

    def qbody(q, carry):
        qxv = plsc.load_gather(qv, [jnp.full((L,), 3 * q, jnp.int32)])
        qyv = plsc.load_gather(qv, [jnp.full((L,), 3 * q + 1, jnp.int32)])
        qzv = plsc.load_gather(qv, [jnp.full((L,), 3 * q + 2, jnp.int32)])
        q2v = (qxv * qxv + qyv * qyv) + qzv * qzv
        qxb = _rne_bf16(qxv)
        qyb = _rne_bf16(qyv)
        qzb = _rne_bf16(qzv)

        def abody(s4, carry):
            tris = list(carry)
            base0 = s4 * (4 * L)
            for u in range(4):
                base = base0 + u * L
                px = pb[pl.ds(base, L)]
                py = pb[pl.ds(N + base, L)]
                pz = pb[pl.ds(2 * N + base, L)]
                x2 = x2v[pl.ds(base, L)]
                inner = px * qxb + py * qyb + pz * qzb
                d2 = (q2v - (inner + inner)) + x2
                dstore[pl.ds(base, L)] = d2
                tris[3 * u], tris[3 * u + 1], tris[3 * u + 2] = _ins3(
                    tris[3 * u], tris[3 * u + 1], tris[3 * u + 2], d2)
            return tuple(tris)

        tris = lax.fori_loop(0, NSTEPS // 4, abody, (inf16,) * 12)
        m1, m2, m3 = tris[0], tris[1], tris[2]
        for u in range(1, 4):
            for x in (tris[3 * u], tris[3 * u + 1], tris[3 * u + 2]):
                m1, m2, m3 = _ins3(m1, m2, m3, x)
        srt = jnp.sort(m3)
        t0v = _take16(srt, jnp.full((L, 1), 10, jnp.int32))

        for r in range(WD):
            wd[pl.ds(r * L, L)] = inf16

        def bbody(s4, cntv):
            base0 = s4 * (4 * L)
            for u in range(4):
                base = base0 + u * L
                d2 = dstore[pl.ds(base, L)]
                msk = d2 <= t0v
                pos = laneoff + jnp.minimum(cntv, WD - 1)
                plsc.store_scatter(wd, [pos], d2, mask=msk)
                plsc.store_scatter(wi, [pos], iota + base, mask=msk)
                cntv = cntv + msk.astype(jnp.int32)
            return cntv

        cntv = lax.fori_loop(0, NSTEPS // 4, bbody, zero16i)
        cntmax = jnp.max(cntv)

        @pl.when(cntmax <= WD)
        def _():
            quads = []
            for qd in range(4):
                runs = []
                for r in range(4):
                    rr = qd * 4 + r
                    runs.append(_sort16(wd[pl.ds(rr * L, L)],
                                        wi[pl.ds(rr * L, L)]))
                A = _merge32(*runs[0], *runs[1], net)
                Bm = _merge32(*runs[2], *runs[3], net)
                quads.append(_low32pair(*A, *Bm, net))
            AB = _low32pair(*quads[0], *quads[1], net)
            CD = _low32pair(*quads[2], *quads[3], net)
            n0k, n0v, n1k, n1v = _low32pair(*AB, *CD, net)
            idxb[pl.ds(q * NS, L)] = n0v
            idxb[pl.ds(q * NS + L, L)] = n1v

        @pl.when(cntmax > WD)
        def _():
            def fb(s, carry):
                k0, v0, k1, v1 = carry
                base = s * L
                dk = dstore[pl.ds(base, L)]
                dv = iota + base
                dk, dv = _sort16(dk, dv)
                rdk = jnp.flip(dk, 0)
                rdv = jnp.flip(dv, 0)
                lk, lv, _, _ = _cmp_ex(k1, v1, rdk, rdv)
                lk, lv = _bsort16(lk, lv, net)
                return _merge32(k0, v0, lk, lv, net)

            k0, v0, k1, v1 = lax.fori_loop(
                0, NSTEPS, fb, (inf16, zero16i, inf16, zero16i))
            idxb[pl.ds(q * NS, L)] = v0
            idxb[pl.ds(q * NS + L, L)] = v1

        return carry

    lax.fori_loop(0, MCHUNK, qbody, 0)

    for dch in range(3):
        def gbody(g, c, dch=dch):
            iv = idxb[pl.ds(g * L, L)]
            vals = plsc.load_gather(pts, [iv + dch * N])
            ob[pl.ds(g * L, L)] = vals
            qc = plsc.load_gather(
                qv, [jnp.full((L,), (g // 2) * 3 + dch, jnp.int32)])
            ob2[pl.ds(g * L, L)] = vals - qc
            return c

        lax.fori_loop(0, NGRP, gbody, 0)
        pltpu.sync_copy(
            ob, gx.at[pl.ds(((b * 3 + dch) * M + mlo) * NS, MCHUNK * NS)])
        pltpu.sync_copy(
            ob2, nf.at[pl.ds(((b * CH + dch) * M + mlo) * NS, MCHUNK * NS)])

    NP = C // 2
    HB = 2 * N

    def _feat_src(pair):
        return features.at[pl.ds((b * C + 2 * pair) * N, HB)]

    def _gather_pair(base, pair):
        def gb(g2, c):
            for uu in range(2):
                g = g2 * 2 + uu
                iv = idxb[pl.ds(g * L, L)]
                ob[pl.ds(g * L, L)] = plsc.load_gather(pb, [iv + base])
                ob2[pl.ds(g * L, L)] = plsc.load_gather(pb, [iv + (base + N)])
            return c

        lax.fori_loop(0, NGRP // 2, gb, 0)
        off = ((b * CH + 3 + 2 * pair) * M + mlo) * NS
        pltpu.sync_copy(ob, nf.at[pl.ds(off, MCHUNK * NS)])
        pltpu.sync_copy(ob2, nf.at[pl.ds(off + M * NS, MCHUNK * NS)])

    pltpu.async_copy(_feat_src(0), pb.at[pl.ds(0, HB)], sem0)

    def fbody(t, carry):
        pa = 2 * t
        pltpu.make_async_copy(_feat_src(pa), pb.at[pl.ds(0, HB)], sem0).wait()
        pltpu.async_copy(_feat_src(pa + 1), pb.at[pl.ds(HB, HB)], sem1)
        _gather_pair(0, pa)
        pltpu.make_async_copy(_feat_src(pa + 1), pb.at[pl.ds(HB, HB)],
                              sem1).wait()
        nxt = jnp.minimum(pa + 2, NP - 1)
        pltpu.async_copy(_feat_src(nxt), pb.at[pl.ds(0, HB)], sem0)
        _gather_pair(HB, pa + 1)
        return carry

    lax.fori_loop(0, NP // 2, fbody, 0)
    pltpu.make_async_copy(_feat_src(NP - 1), pb.at[pl.ds(0, HB)], sem0).wait()


def kernel(xyz, new_xyz, features):
    xyzT = jnp.transpose(xyz, (0, 2, 1)).reshape(-1)
    mesh = plsc.VectorSubcoreMesh(core_axis_name="c", subcore_axis_name="s",
                                  num_cores=2, num_subcores=16)
    nf, gx = pl.kernel(
        _body,
        out_type=[
            jax.ShapeDtypeStruct((B * CH * M * NS,), jnp.float32),
            jax.ShapeDtypeStruct((B * 3 * M * NS,), jnp.float32),
        ],
        mesh=mesh,
        compiler_params=pltpu.CompilerParams(needs_layout_passes=False),
        scratch_types=[
            pltpu.VMEM((3 * N,), jnp.float32),
            pltpu.VMEM((4 * N,), jnp.float32),
            pltpu.VMEM((N,), jnp.float32),
            pltpu.VMEM((N,), jnp.float32),
            pltpu.VMEM((MCHUNK * 3,), jnp.float32),
            pltpu.VMEM((WD * L,), jnp.float32),
            pltpu.VMEM((WD * L,), jnp.int32),
            pltpu.VMEM((MCHUNK * NS,), jnp.int32),
            pltpu.VMEM((MCHUNK * NS,), jnp.float32),
            pltpu.VMEM((MCHUNK * NS,), jnp.float32),
            pltpu.SemaphoreType.DMA,
            pltpu.SemaphoreType.DMA,
        ],
    )(xyzT, new_xyz.reshape(-1), features.reshape(-1))
    return (nf.reshape(B, CH, M, NS), gx.reshape(B, 3, M, NS))


# speedup vs baseline: 12.8467x; 1.0145x over previous
import jax
import jax.numpy as jnp
from jax import lax
from jax.experimental import pallas as pl
from jax.experimental.pallas import tpu as pltpu
from jax.experimental.pallas import tpu_sc as plsc

B, N, M, C, NS = 4, 8192, 2048, 96, 32
L = 16
NW = 32
TPB = NW // B
MCHUNK = M // TPB
NSTEPS = N // L
NGRP = MCHUNK * NS // L
CH = 3 + C
WD = 16


_GDN = lax.GatherDimensionNumbers(offset_dims=(), collapsed_slice_dims=(0,),
                                  start_index_map=(0,))


def _take16(x, idxcol):
    return lax.gather(x, idxcol, _GDN, (1,),
                      mode=lax.GatherScatterMode.PROMISE_IN_BOUNDS)


def _cmp_ex(ka, va, kb, vb):
    m = (ka < kb) | ((ka == kb) & (va <= vb))
    lo_k = jnp.where(m, ka, kb)
    lo_v = jnp.where(m, va, vb)
    hi_k = jnp.where(m, kb, ka)
    hi_v = jnp.where(m, vb, va)
    return lo_k, lo_v, hi_k, hi_v


def _sort16(k, v):
    r = plsc.sort_key_val(k, v)
    return r[0], r[1]


def _bsort16(k, v, net):
    for idxcol, low in net:
        pk = _take16(k, idxcol)
        pv = _take16(v, idxcol)
        m = (k < pk) | ((k == pk) & (v <= pv))
        kmin = jnp.where(m, k, pk)
        kmax = jnp.where(m, pk, k)
        vmin = jnp.where(m, v, pv)
        vmax = jnp.where(m, pv, v)
        k = jnp.where(low, kmin, kmax)
        v = jnp.where(low, vmin, vmax)
    return k, v


def _merge32(ka, va, kb, vb, net):
    rkb = jnp.flip(kb, 0)
    rvb = jnp.flip(vb, 0)
    lk, lv, hk, hv = _cmp_ex(ka, va, rkb, rvb)
    lk, lv = _bsort16(lk, lv, net)
    hk, hv = _bsort16(hk, hv, net)
    return lk, lv, hk, hv


def _low32pair(ak0, av0, ak1, av1, bk0, bv0, bk1, bv1, net):
    rbk0 = jnp.flip(bk1, 0)
    rbv0 = jnp.flip(bv1, 0)
    rbk1 = jnp.flip(bk0, 0)
    rbv1 = jnp.flip(bv0, 0)
    l0k, l0v, _, _ = _cmp_ex(ak0, av0, rbk0, rbv0)
    l1k, l1v, _, _ = _cmp_ex(ak1, av1, rbk1, rbv1)
    n0k, n0v, n1k, n1v = _cmp_ex(l0k, l0v, l1k, l1v)
    n0k, n0v = _bsort16(n0k, n0v, net)
    n1k, n1v = _bsort16(n1k, n1v, net)
    return n0k, n0v, n1k, n1v


def _ins3(m1, m2, m3, x):
    a = jnp.maximum(m1, x)
    m1 = jnp.minimum(m1, x)
    bv = jnp.maximum(m2, a)
    m2 = jnp.minimum(m2, a)
    m3 = jnp.minimum(m3, bv)
    return m1, m2, m3


def _rne_bf16(x):
    u = plsc.bitcast(x, jnp.int32)
    r = u + 0x7FFF + ((u >> 16) & 1)
    r = r & jnp.int32(-65536)
    return plsc.bitcast(r, jnp.float32)


def _body(xyzT, new_xyz, features, nf, gx,
          pts, pb, x2v, dstore, qv, wd, wi, idxb, ob, ob2, ob3, ob4,
          sem0, sem1, sem2):
    wid = lax.axis_index("s") * 2 + lax.axis_index("c")
    b = wid // TPB
    mlo = (wid % TPB) * MCHUNK

    pltpu.sync_copy(xyzT.at[pl.ds(b * 3 * N, 3 * N)], pts)
    pltpu.sync_copy(new_xyz.at[pl.ds((b * M + mlo) * 3, MCHUNK * 3)], qv)

    iota = lax.iota(jnp.int32, L)
    inf16 = jnp.full((L,), jnp.inf, jnp.float32)
    zero16i = jnp.zeros((L,), jnp.int32)
    laneoff = iota * WD
    net = tuple((jnp.reshape(iota ^ s, (L, 1)), (iota & s) == 0)
                for s in (8, 4, 2, 1))

    def pbody(s, c):
        base = s * L
        px = pts[pl.ds(base, L)]
        py = pts[pl.ds(N + base, L)]
        pz = pts[pl.ds(2 * N + base, L)]
        pb[pl.ds(base, L)] = _rne_bf16(px)
        pb[pl.ds(N + base, L)] = _rne_bf16(py)
        pb[pl.ds(2 * N + base, L)] = _rne_bf16(pz)
        x2v[pl.ds(base, L)] = (px * px + py * py) + pz * pz
        return c

    lax.fori_loop(0, NSTEPS, pbody, 0)

    def qbody(q, carry):
        qxv = plsc.load_gather(qv, [jnp.full((L,), 3 * q, jnp.int32)])
        qyv = plsc.load_gather(qv, [jnp.full((L,), 3 * q + 1, jnp.int32)])
        qzv = plsc.load_gather(qv, [jnp.full((L,), 3 * q + 2, jnp.int32)])
        q2v = (qxv * qxv + qyv * qyv) + qzv * qzv
        qxb = _rne_bf16(qxv)
        qyb = _rne_bf16(qyv)
        qzb = _rne_bf16(qzv)

        def abody(s4, carry):
            tris = list(carry)
            base0 = s4 * (4 * L)
            for u in range(4):
                base = base0 + u * L
                px = pb[pl.ds(base, L)]
                py = pb[pl.ds(N + base, L)]
                pz = pb[pl.ds(2 * N + base, L)]
                x2 = x2v[pl.ds(base, L)]
                inner = px * qxb + py * qyb + pz * qzb
                d2 = (q2v - (inner + inner)) + x2
                dstore[pl.ds(base, L)] = d2
                tris[3 * u], tris[3 * u + 1], tris[3 * u + 2] = _ins3(
                    tris[3 * u], tris[3 * u + 1], tris[3 * u + 2], d2)
            return tuple(tris)

        tris = lax.fori_loop(0, NSTEPS // 4, abody, (inf16,) * 12)
        m1, m2, m3 = tris[0], tris[1], tris[2]
        for u in range(1, 4):
            for x in (tris[3 * u], tris[3 * u + 1], tris[3 * u + 2]):
                m1, m2, m3 = _ins3(m1, m2, m3, x)
        srt = jnp.sort(m3)
        t0v = _take16(srt, jnp.full((L, 1), 10, jnp.int32))

        for r in range(WD):
            wd[pl.ds(r * L, L)] = inf16

        def bbody(s4, cntv):
            base0 = s4 * (4 * L)
            for u in range(4):
                base = base0 + u * L
                d2 = dstore[pl.ds(base, L)]
                msk = d2 <= t0v
                pos = laneoff + jnp.minimum(cntv, WD - 1)
                plsc.store_scatter(wd, [pos], d2, mask=msk)
                plsc.store_scatter(wi, [pos], iota + base, mask=msk)
                cntv = cntv + msk.astype(jnp.int32)
            return cntv

        cntv = lax.fori_loop(0, NSTEPS // 4, bbody, zero16i)
        cntmax = jnp.max(cntv)

        @pl.when(cntmax <= WD)
        def _():
            quads = []
            for qd in range(4):
                runs = []
                for r in range(4):
                    rr = qd * 4 + r
                    runs.append(_sort16(wd[pl.ds(rr * L, L)],
                                        wi[pl.ds(rr * L, L)]))
                A = _merge32(*runs[0], *runs[1], net)
                Bm = _merge32(*runs[2], *runs[3], net)
                quads.append(_low32pair(*A, *Bm, net))
            AB = _low32pair(*quads[0], *quads[1], net)
            CD = _low32pair(*quads[2], *quads[3], net)
            n0k, n0v, n1k, n1v = _low32pair(*AB, *CD, net)
            idxb[pl.ds(q * NS, L)] = n0v
            idxb[pl.ds(q * NS + L, L)] = n1v

        @pl.when(cntmax > WD)
        def _():
            def fb(s, carry):
                k0, v0, k1, v1 = carry
                base = s * L
                dk = dstore[pl.ds(base, L)]
                dv = iota + base
                dk, dv = _sort16(dk, dv)
                rdk = jnp.flip(dk, 0)
                rdv = jnp.flip(dv, 0)
                lk, lv, _, _ = _cmp_ex(k1, v1, rdk, rdv)
                lk, lv = _bsort16(lk, lv, net)
                return _merge32(k0, v0, lk, lv, net)

            k0, v0, k1, v1 = lax.fori_loop(
                0, NSTEPS, fb, (inf16, zero16i, inf16, zero16i))
            idxb[pl.ds(q * NS, L)] = v0
            idxb[pl.ds(q * NS + L, L)] = v1

        return carry

    lax.fori_loop(0, MCHUNK, qbody, 0)

    for dch in range(3):
        def gbody(g, c, dch=dch):
            iv = idxb[pl.ds(g * L, L)]
            vals = plsc.load_gather(pts, [iv + dch * N])
            ob[pl.ds(g * L, L)] = vals
            qc = plsc.load_gather(
                qv, [jnp.full((L,), (g // 2) * 3 + dch, jnp.int32)])
            ob2[pl.ds(g * L, L)] = vals - qc
            return c

        lax.fori_loop(0, NGRP, gbody, 0)
        pltpu.sync_copy(
            ob, gx.at[pl.ds(((b * 3 + dch) * M + mlo) * NS, MCHUNK * NS)])
        pltpu.sync_copy(
            ob2, nf.at[pl.ds(((b * CH + dch) * M + mlo) * NS, MCHUNK * NS)])

    NP = C // 2
    HB = 2 * N

    def _feat_src(pair):
        return features.at[pl.ds((b * C + 2 * pair) * N, HB)]

    OBH = MCHUNK * NS

    def _ob_dst(pair):
        off = ((b * CH + 3 + 2 * pair) * M + mlo) * NS
        return (nf.at[pl.ds(off, OBH)], nf.at[pl.ds(off + M * NS, OBH)])

    def _gather_pair(base, bufa, bufb, pair):
        def gb(g2, c):
            for uu in range(2):
                g = g2 * 2 + uu
                iv = idxb[pl.ds(g * L, L)]
                bufa[pl.ds(g * L, L)] = plsc.load_gather(pb, [iv + base])
                bufb[pl.ds(g * L, L)] = plsc.load_gather(pb, [iv + (base + N)])
            return c

        lax.fori_loop(0, NGRP // 2, gb, 0)
        d0, d1 = _ob_dst(pair)
        pltpu.async_copy(bufa, d0, sem2)
        pltpu.async_copy(bufb, d1, sem2)

    def _wait_out(bufa, bufb, pair):
        d0, d1 = _ob_dst(pair)
        pltpu.make_async_copy(bufa, d0, sem2).wait()
        pltpu.make_async_copy(bufb, d1, sem2).wait()

    pltpu.async_copy(_feat_src(0), pb.at[pl.ds(0, HB)], sem0)

    def fbody(t, carry):
        pa = 2 * t
        pltpu.make_async_copy(_feat_src(pa), pb.at[pl.ds(0, HB)], sem0).wait()
        pltpu.async_copy(_feat_src(pa + 1), pb.at[pl.ds(HB, HB)], sem1)

        @pl.when(t > 0)
        def _():
            _wait_out(ob, ob2, pa - 2)

        _gather_pair(0, ob, ob2, pa)
        pltpu.make_async_copy(_feat_src(pa + 1), pb.at[pl.ds(HB, HB)],
                              sem1).wait()
        nxt = jnp.minimum(pa + 2, NP - 1)
        pltpu.async_copy(_feat_src(nxt), pb.at[pl.ds(0, HB)], sem0)

        @pl.when(t > 0)
        def _():
            _wait_out(ob3, ob4, pa - 1)

        _gather_pair(HB, ob3, ob4, pa + 1)
        return carry

    lax.fori_loop(0, NP // 2, fbody, 0)
    pltpu.make_async_copy(_feat_src(NP - 1), pb.at[pl.ds(0, HB)], sem0).wait()
    _wait_out(ob, ob2, NP - 2)
    _wait_out(ob3, ob4, NP - 1)


def kernel(xyz, new_xyz, features):
    xyzT = jnp.transpose(xyz, (0, 2, 1)).reshape(-1)
    mesh = plsc.VectorSubcoreMesh(core_axis_name="c", subcore_axis_name="s",
                                  num_cores=2, num_subcores=16)
    nf, gx = pl.kernel(
        _body,
        out_type=[
            jax.ShapeDtypeStruct((B * CH * M * NS,), jnp.float32),
            jax.ShapeDtypeStruct((B * 3 * M * NS,), jnp.float32),
        ],
        mesh=mesh,
        compiler_params=pltpu.CompilerParams(needs_layout_passes=False),
        scratch_types=[
            pltpu.VMEM((3 * N,), jnp.float32),
            pltpu.VMEM((4 * N,), jnp.float32),
            pltpu.VMEM((N,), jnp.float32),
            pltpu.VMEM((N,), jnp.float32),
            pltpu.VMEM((MCHUNK * 3,), jnp.float32),
            pltpu.VMEM((WD * L,), jnp.float32),
            pltpu.VMEM((WD * L,), jnp.int32),
            pltpu.VMEM((MCHUNK * NS,), jnp.int32),
            pltpu.VMEM((MCHUNK * NS,), jnp.float32),
            pltpu.VMEM((MCHUNK * NS,), jnp.float32),
            pltpu.VMEM((MCHUNK * NS,), jnp.float32),
            pltpu.VMEM((MCHUNK * NS,), jnp.float32),
            pltpu.SemaphoreType.DMA,
            pltpu.SemaphoreType.DMA,
            pltpu.SemaphoreType.DMA,
        ],
    )(xyzT, new_xyz.reshape(-1), features.reshape(-1))
    return (nf.reshape(B, CH, M, NS), gx.reshape(B, 3, M, NS))


# speedup vs baseline: 12.9829x; 1.0106x over previous
import jax
import jax.numpy as jnp
from jax import lax
from jax.experimental import pallas as pl
from jax.experimental.pallas import tpu as pltpu
from jax.experimental.pallas import tpu_sc as plsc

B, N, M, C, NS = 4, 8192, 2048, 96, 32
L = 16
NW = 32
TPB = NW // B
MCHUNK = M // TPB
NSTEPS = N // L
NGRP = MCHUNK * NS // L
CH = 3 + C
WD = 16


_GDN = lax.GatherDimensionNumbers(offset_dims=(), collapsed_slice_dims=(0,),
                                  start_index_map=(0,))


def _take16(x, idxcol):
    return lax.gather(x, idxcol, _GDN, (1,),
                      mode=lax.GatherScatterMode.PROMISE_IN_BOUNDS)


def _cmp_ex(ka, va, kb, vb):
    m = (ka < kb) | ((ka == kb) & (va <= vb))
    lo_k = jnp.where(m, ka, kb)
    lo_v = jnp.where(m, va, vb)
    hi_k = jnp.where(m, kb, ka)
    hi_v = jnp.where(m, vb, va)
    return lo_k, lo_v, hi_k, hi_v


def _sort16(k, v):
    r = plsc.sort_key_val(k, v)
    return r[0], r[1]


def _bsort16(k, v, net):
    for idxcol, low in net:
        pk = _take16(k, idxcol)
        pv = _take16(v, idxcol)
        m = (k < pk) | ((k == pk) & (v <= pv))
        kmin = jnp.where(m, k, pk)
        kmax = jnp.where(m, pk, k)
        vmin = jnp.where(m, v, pv)
        vmax = jnp.where(m, pv, v)
        k = jnp.where(low, kmin, kmax)
        v = jnp.where(low, vmin, vmax)
    return k, v


def _merge32(ka, va, kb, vb, net):
    rkb = jnp.flip(kb, 0)
    rvb = jnp.flip(vb, 0)
    lk, lv, hk, hv = _cmp_ex(ka, va, rkb, rvb)
    lk, lv = _bsort16(lk, lv, net)
    hk, hv = _bsort16(hk, hv, net)
    return lk, lv, hk, hv


def _low32pair(ak0, av0, ak1, av1, bk0, bv0, bk1, bv1, net):
    rbk0 = jnp.flip(bk1, 0)
    rbv0 = jnp.flip(bv1, 0)
    rbk1 = jnp.flip(bk0, 0)
    rbv1 = jnp.flip(bv0, 0)
    l0k, l0v, _, _ = _cmp_ex(ak0, av0, rbk0, rbv0)
    l1k, l1v, _, _ = _cmp_ex(ak1, av1, rbk1, rbv1)
    n0k, n0v, n1k, n1v = _cmp_ex(l0k, l0v, l1k, l1v)
    n0k, n0v = _bsort16(n0k, n0v, net)
    n1k, n1v = _bsort16(n1k, n1v, net)
    return n0k, n0v, n1k, n1v


def _ins3(m1, m2, m3, x):
    a = jnp.maximum(m1, x)
    m1 = jnp.minimum(m1, x)
    bv = jnp.maximum(m2, a)
    m2 = jnp.minimum(m2, a)
    m3 = jnp.minimum(m3, bv)
    return m1, m2, m3


def _rne_bf16(x):
    u = plsc.bitcast(x, jnp.int32)
    r = u + 0x7FFF + ((u >> 16) & 1)
    r = r & jnp.int32(-65536)
    return plsc.bitcast(r, jnp.float32)


def _body(xyzT, new_xyz, features, nf, gx,
          pts, pb, x2v, dstore, qv, wd, wi, idxb, ob, ob2, ob3, ob4,
          sem0, sem1, sem2):
    wid = lax.axis_index("s") * 2 + lax.axis_index("c")
    b = wid // TPB
    mlo = (wid % TPB) * MCHUNK

    pltpu.sync_copy(xyzT.at[pl.ds(b * 3 * N, 3 * N)], pts)
    pltpu.sync_copy(new_xyz.at[pl.ds((b * M + mlo) * 3, MCHUNK * 3)], qv)

    iota = lax.iota(jnp.int32, L)
    inf16 = jnp.full((L,), jnp.inf, jnp.float32)
    zero16i = jnp.zeros((L,), jnp.int32)
    laneoff = iota * WD
    net = tuple((jnp.reshape(iota ^ s, (L, 1)), (iota & s) == 0)
                for s in (8, 4, 2, 1))

    def pbody(s, c):
        base = s * L
        px = pts[pl.ds(base, L)]
        py = pts[pl.ds(N + base, L)]
        pz = pts[pl.ds(2 * N + base, L)]
        pb[pl.ds(base, L)] = _rne_bf16(px)
        pb[pl.ds(N + base, L)] = _rne_bf16(py)
        pb[pl.ds(2 * N + base, L)] = _rne_bf16(pz)
        x2v[pl.ds(base, L)] = (px * px + py * py) + pz * pz
        return c

    lax.fori_loop(0, NSTEPS, pbody, 0)

    def qbody(q, carry):
        qxv = plsc.load_gather(qv, [jnp.full((L,), 3 * q, jnp.int32)])
        qyv = plsc.load_gather(qv, [jnp.full((L,), 3 * q + 1, jnp.int32)])
        qzv = plsc.load_gather(qv, [jnp.full((L,), 3 * q + 2, jnp.int32)])
        q2v = (qxv * qxv + qyv * qyv) + qzv * qzv
        qxb = _rne_bf16(qxv)
        qyb = _rne_bf16(qyv)
        qzb = _rne_bf16(qzv)

        def abody(s8, carry):
            tris = list(carry)
            base0 = s8 * (8 * L)
            for u in range(8):
                base = base0 + u * L
                px = pb[pl.ds(base, L)]
                py = pb[pl.ds(N + base, L)]
                pz = pb[pl.ds(2 * N + base, L)]
                x2 = x2v[pl.ds(base, L)]
                inner = px * qxb + py * qyb + pz * qzb
                d2 = (q2v - (inner + inner)) + x2
                dstore[pl.ds(base, L)] = d2
                t = 3 * (u % 4)
                tris[t], tris[t + 1], tris[t + 2] = _ins3(
                    tris[t], tris[t + 1], tris[t + 2], d2)
            return tuple(tris)

        tris = lax.fori_loop(0, NSTEPS // 8, abody, (inf16,) * 12)
        m1, m2, m3 = tris[0], tris[1], tris[2]
        for u in range(1, 4):
            for x in (tris[3 * u], tris[3 * u + 1], tris[3 * u + 2]):
                m1, m2, m3 = _ins3(m1, m2, m3, x)
        srt = jnp.sort(m3)
        t0v = _take16(srt, jnp.full((L, 1), 10, jnp.int32))

        for r in range(WD):
            wd[pl.ds(r * L, L)] = inf16

        def bbody(s8, cntv):
            base0 = s8 * (8 * L)
            for u in range(8):
                base = base0 + u * L
                d2 = dstore[pl.ds(base, L)]
                msk = d2 <= t0v
                pos = laneoff + jnp.minimum(cntv, WD - 1)
                plsc.store_scatter(wd, [pos], d2, mask=msk)
                plsc.store_scatter(wi, [pos], iota + base, mask=msk)
                cntv = cntv + msk.astype(jnp.int32)
            return cntv

        cntv = lax.fori_loop(0, NSTEPS // 8, bbody, zero16i)
        cntmax = jnp.max(cntv)

        @pl.when(cntmax <= WD)
        def _():
            quads = []
            for qd in range(4):
                runs = []
                for r in range(4):
                    rr = qd * 4 + r
                    runs.append(_sort16(wd[pl.ds(rr * L, L)],
                                        wi[pl.ds(rr * L, L)]))
                A = _merge32(*runs[0], *runs[1], net)
                Bm = _merge32(*runs[2], *runs[3], net)
                quads.append(_low32pair(*A, *Bm, net))
            AB = _low32pair(*quads[0], *quads[1], net)
            CD = _low32pair(*quads[2], *quads[3], net)
            n0k, n0v, n1k, n1v = _low32pair(*AB, *CD, net)
            idxb[pl.ds(q * NS, L)] = n0v
            idxb[pl.ds(q * NS + L, L)] = n1v

        @pl.when(cntmax > WD)
        def _():
            def fb(s, carry):
                k0, v0, k1, v1 = carry
                base = s * L
                dk = dstore[pl.ds(base, L)]
                dv = iota + base
                dk, dv = _sort16(dk, dv)
                rdk = jnp.flip(dk, 0)
                rdv = jnp.flip(dv, 0)
                lk, lv, _, _ = _cmp_ex(k1, v1, rdk, rdv)
                lk, lv = _bsort16(lk, lv, net)
                return _merge32(k0, v0, lk, lv, net)

            k0, v0, k1, v1 = lax.fori_loop(
                0, NSTEPS, fb, (inf16, zero16i, inf16, zero16i))
            idxb[pl.ds(q * NS, L)] = v0
            idxb[pl.ds(q * NS + L, L)] = v1

        return carry

    lax.fori_loop(0, MCHUNK, qbody, 0)

    for dch in range(3):
        def gbody(g, c, dch=dch):
            iv = idxb[pl.ds(g * L, L)]
            vals = plsc.load_gather(pts, [iv + dch * N])
            ob[pl.ds(g * L, L)] = vals
            qc = plsc.load_gather(
                qv, [jnp.full((L,), (g // 2) * 3 + dch, jnp.int32)])
            ob2[pl.ds(g * L, L)] = vals - qc
            return c

        lax.fori_loop(0, NGRP, gbody, 0)
        pltpu.sync_copy(
            ob, gx.at[pl.ds(((b * 3 + dch) * M + mlo) * NS, MCHUNK * NS)])
        pltpu.sync_copy(
            ob2, nf.at[pl.ds(((b * CH + dch) * M + mlo) * NS, MCHUNK * NS)])

    NP = C // 2
    HB = 2 * N

    def _feat_src(pair):
        return features.at[pl.ds((b * C + 2 * pair) * N, HB)]

    OBH = MCHUNK * NS

    def _ob_dst(pair):
        off = ((b * CH + 3 + 2 * pair) * M + mlo) * NS
        return (nf.at[pl.ds(off, OBH)], nf.at[pl.ds(off + M * NS, OBH)])

    def _gather_pair(base, bufa, bufb, pair):
        def gb(g2, c):
            for uu in range(2):
                g = g2 * 2 + uu
                iv = idxb[pl.ds(g * L, L)]
                bufa[pl.ds(g * L, L)] = plsc.load_gather(pb, [iv + base])
                bufb[pl.ds(g * L, L)] = plsc.load_gather(pb, [iv + (base + N)])
            return c

        lax.fori_loop(0, NGRP // 2, gb, 0)
        d0, d1 = _ob_dst(pair)
        pltpu.async_copy(bufa, d0, sem2)
        pltpu.async_copy(bufb, d1, sem2)

    def _wait_out(bufa, bufb, pair):
        d0, d1 = _ob_dst(pair)
        pltpu.make_async_copy(bufa, d0, sem2).wait()
        pltpu.make_async_copy(bufb, d1, sem2).wait()

    pltpu.async_copy(_feat_src(0), pb.at[pl.ds(0, HB)], sem0)

    def fbody(t, carry):
        pa = 2 * t
        pltpu.make_async_copy(_feat_src(pa), pb.at[pl.ds(0, HB)], sem0).wait()
        pltpu.async_copy(_feat_src(pa + 1), pb.at[pl.ds(HB, HB)], sem1)

        @pl.when(t > 0)
        def _():
            _wait_out(ob, ob2, pa - 2)

        _gather_pair(0, ob, ob2, pa)
        pltpu.make_async_copy(_feat_src(pa + 1), pb.at[pl.ds(HB, HB)],
                              sem1).wait()
        nxt = jnp.minimum(pa + 2, NP - 1)
        pltpu.async_copy(_feat_src(nxt), pb.at[pl.ds(0, HB)], sem0)

        @pl.when(t > 0)
        def _():
            _wait_out(ob3, ob4, pa - 1)

        _gather_pair(HB, ob3, ob4, pa + 1)
        return carry

    lax.fori_loop(0, NP // 2, fbody, 0)
    pltpu.make_async_copy(_feat_src(NP - 1), pb.at[pl.ds(0, HB)], sem0).wait()
    _wait_out(ob, ob2, NP - 2)
    _wait_out(ob3, ob4, NP - 1)


def kernel(xyz, new_xyz, features):
    xyzT = jnp.transpose(xyz, (0, 2, 1)).reshape(-1)
    mesh = plsc.VectorSubcoreMesh(core_axis_name="c", subcore_axis_name="s",
                                  num_cores=2, num_subcores=16)
    nf, gx = pl.kernel(
        _body,
        out_type=[
            jax.ShapeDtypeStruct((B * CH * M * NS,), jnp.float32),
            jax.ShapeDtypeStruct((B * 3 * M * NS,), jnp.float32),
        ],
        mesh=mesh,
        compiler_params=pltpu.CompilerParams(needs_layout_passes=False),
        scratch_types=[
            pltpu.VMEM((3 * N,), jnp.float32),
            pltpu.VMEM((4 * N,), jnp.float32),
            pltpu.VMEM((N,), jnp.float32),
            pltpu.VMEM((N,), jnp.float32),
            pltpu.VMEM((MCHUNK * 3,), jnp.float32),
            pltpu.VMEM((WD * L,), jnp.float32),
            pltpu.VMEM((WD * L,), jnp.int32),
            pltpu.VMEM((MCHUNK * NS,), jnp.int32),
            pltpu.VMEM((MCHUNK * NS,), jnp.float32),
            pltpu.VMEM((MCHUNK * NS,), jnp.float32),
            pltpu.VMEM((MCHUNK * NS,), jnp.float32),
            pltpu.VMEM((MCHUNK * NS,), jnp.float32),
            pltpu.SemaphoreType.DMA,
            pltpu.SemaphoreType.DMA,
            pltpu.SemaphoreType.DMA,
        ],
    )(xyzT, new_xyz.reshape(-1), features.reshape(-1))
    return (nf.reshape(B, CH, M, NS), gx.reshape(B, 3, M, NS))


# speedup vs baseline: 13.4249x; 1.0340x over previous
import jax
import jax.numpy as jnp
from jax import lax
from jax.experimental import pallas as pl
from jax.experimental.pallas import tpu as pltpu
from jax.experimental.pallas import tpu_sc as plsc

B, N, M, C, NS = 4, 8192, 2048, 96, 32
L = 16
NW = 32
TPB = NW // B
MCHUNK = M // TPB
NSTEPS = N // L
NGRP = MCHUNK * NS // L
CH = 3 + C
WD = 16


_GDN = lax.GatherDimensionNumbers(offset_dims=(), collapsed_slice_dims=(0,),
                                  start_index_map=(0,))


def _take16(x, idxcol):
    return lax.gather(x, idxcol, _GDN, (1,),
                      mode=lax.GatherScatterMode.PROMISE_IN_BOUNDS)


def _cmp_ex(ka, va, kb, vb):
    m = (ka < kb) | ((ka == kb) & (va <= vb))
    lo_k = jnp.where(m, ka, kb)
    lo_v = jnp.where(m, va, vb)
    hi_k = jnp.where(m, kb, ka)
    hi_v = jnp.where(m, vb, va)
    return lo_k, lo_v, hi_k, hi_v


def _sort16(k, v):
    r = plsc.sort_key_val(k, v)
    return r[0], r[1]


def _bsort16(k, v, net):
    for idxcol, low in net:
        pk = _take16(k, idxcol)
        pv = _take16(v, idxcol)
        m = (k < pk) | ((k == pk) & (v <= pv))
        kmin = jnp.where(m, k, pk)
        kmax = jnp.where(m, pk, k)
        vmin = jnp.where(m, v, pv)
        vmax = jnp.where(m, pv, v)
        k = jnp.where(low, kmin, kmax)
        v = jnp.where(low, vmin, vmax)
    return k, v


def _merge32(ka, va, kb, vb, net):
    rkb = jnp.flip(kb, 0)
    rvb = jnp.flip(vb, 0)
    lk, lv, hk, hv = _cmp_ex(ka, va, rkb, rvb)
    lk, lv = _bsort16(lk, lv, net)
    hk, hv = _bsort16(hk, hv, net)
    return lk, lv, hk, hv


def _low32pair(ak0, av0, ak1, av1, bk0, bv0, bk1, bv1, net):
    rbk0 = jnp.flip(bk1, 0)
    rbv0 = jnp.flip(bv1, 0)
    rbk1 = jnp.flip(bk0, 0)
    rbv1 = jnp.flip(bv0, 0)
    l0k, l0v, _, _ = _cmp_ex(ak0, av0, rbk0, rbv0)
    l1k, l1v, _, _ = _cmp_ex(ak1, av1, rbk1, rbv1)
    n0k, n0v, n1k, n1v = _cmp_ex(l0k, l0v, l1k, l1v)
    n0k, n0v = _bsort16(n0k, n0v, net)
    n1k, n1v = _bsort16(n1k, n1v, net)
    return n0k, n0v, n1k, n1v


def _ins3(m1, m2, m3, x):
    a = jnp.maximum(m1, x)
    m1 = jnp.minimum(m1, x)
    bv = jnp.maximum(m2, a)
    m2 = jnp.minimum(m2, a)
    m3 = jnp.minimum(m3, bv)
    return m1, m2, m3


def _rne_bf16(x):
    u = plsc.bitcast(x, jnp.int32)
    r = u + 0x7FFF + ((u >> 16) & 1)
    r = r & jnp.int32(-65536)
    return plsc.bitcast(r, jnp.float32)


def _body(xyzT, new_xyz, features, nf, gx,
          pts, pb, x2v, dstore, qv, wi, idxb, ob, ob2, ob3, ob4,
          sem0, sem1, sem2):
    wid = lax.axis_index("s") * 2 + lax.axis_index("c")
    b = wid // TPB
    mlo = (wid % TPB) * MCHUNK

    pltpu.sync_copy(xyzT.at[pl.ds(b * 3 * N, 3 * N)], pts)
    pltpu.sync_copy(new_xyz.at[pl.ds((b * M + mlo) * 3, MCHUNK * 3)], qv)

    iota = lax.iota(jnp.int32, L)
    inf16 = jnp.full((L,), jnp.inf, jnp.float32)
    zero16i = jnp.zeros((L,), jnp.int32)
    laneoff = iota * WD
    net = tuple((jnp.reshape(iota ^ s, (L, 1)), (iota & s) == 0)
                for s in (8, 4, 2, 1))

    def pbody(s, c):
        base = s * L
        px = pts[pl.ds(base, L)]
        py = pts[pl.ds(N + base, L)]
        pz = pts[pl.ds(2 * N + base, L)]
        pb[pl.ds(base, L)] = _rne_bf16(px)
        pb[pl.ds(N + base, L)] = _rne_bf16(py)
        pb[pl.ds(2 * N + base, L)] = _rne_bf16(pz)
        x2v[pl.ds(base, L)] = (px * px + py * py) + pz * pz
        return c

    lax.fori_loop(0, NSTEPS, pbody, 0)
    dstore[pl.ds(N, L)] = jnp.full((L,), jnp.inf, jnp.float32)

    def qbody(q, carry):
        qxv = plsc.load_gather(qv, [jnp.full((L,), 3 * q, jnp.int32)])
        qyv = plsc.load_gather(qv, [jnp.full((L,), 3 * q + 1, jnp.int32)])
        qzv = plsc.load_gather(qv, [jnp.full((L,), 3 * q + 2, jnp.int32)])
        q2v = (qxv * qxv + qyv * qyv) + qzv * qzv
        qxb = _rne_bf16(qxv)
        qyb = _rne_bf16(qyv)
        qzb = _rne_bf16(qzv)

        def abody(s8, carry):
            tris = list(carry)
            base0 = s8 * (8 * L)
            for u in range(8):
                base = base0 + u * L
                px = pb[pl.ds(base, L)]
                py = pb[pl.ds(N + base, L)]
                pz = pb[pl.ds(2 * N + base, L)]
                x2 = x2v[pl.ds(base, L)]
                inner = px * qxb + py * qyb + pz * qzb
                d2 = (q2v - (inner + inner)) + x2
                dstore[pl.ds(base, L)] = d2
                t = 3 * (u % 4)
                tris[t], tris[t + 1], tris[t + 2] = _ins3(
                    tris[t], tris[t + 1], tris[t + 2], d2)
            return tuple(tris)

        tris = lax.fori_loop(0, NSTEPS // 8, abody, (inf16,) * 12)
        m1, m2, m3 = tris[0], tris[1], tris[2]
        for u in range(1, 4):
            for x in (tris[3 * u], tris[3 * u + 1], tris[3 * u + 2]):
                m1, m2, m3 = _ins3(m1, m2, m3, x)
        srt = jnp.sort(m3)
        t0v = _take16(srt, jnp.full((L, 1), 10, jnp.int32))

        npad = jnp.full((L,), N, jnp.int32)
        for r in range(WD):
            wi[pl.ds(r * L, L)] = npad

        def bbody(s8, cntv):
            base0 = s8 * (8 * L)
            for u in range(8):
                base = base0 + u * L
                d2 = dstore[pl.ds(base, L)]
                msk = d2 <= t0v
                pos = laneoff + jnp.minimum(cntv, WD - 1)
                plsc.store_scatter(wi, [pos], iota + base, mask=msk)
                cntv = cntv + msk.astype(jnp.int32)
            return cntv

        cntv = lax.fori_loop(0, NSTEPS // 8, bbody, zero16i)
        cntmax = jnp.max(cntv)

        @pl.when(cntmax <= WD)
        def _():
            quads = []
            for qd in range(4):
                runs = []
                for r in range(4):
                    rr = qd * 4 + r
                    ivr = wi[pl.ds(rr * L, L)]
                    kr = plsc.load_gather(dstore, [ivr])
                    runs.append(_sort16(kr, ivr))
                A = _merge32(*runs[0], *runs[1], net)
                Bm = _merge32(*runs[2], *runs[3], net)
                quads.append(_low32pair(*A, *Bm, net))
            AB = _low32pair(*quads[0], *quads[1], net)
            CD = _low32pair(*quads[2], *quads[3], net)
            n0k, n0v, n1k, n1v = _low32pair(*AB, *CD, net)
            idxb[pl.ds(q * NS, L)] = n0v
            idxb[pl.ds(q * NS + L, L)] = n1v

        @pl.when(cntmax > WD)
        def _():
            def fb(s, carry):
                k0, v0, k1, v1 = carry
                base = s * L
                dk = dstore[pl.ds(base, L)]
                dv = iota + base
                dk, dv = _sort16(dk, dv)
                rdk = jnp.flip(dk, 0)
                rdv = jnp.flip(dv, 0)
                lk, lv, _, _ = _cmp_ex(k1, v1, rdk, rdv)
                lk, lv = _bsort16(lk, lv, net)
                return _merge32(k0, v0, lk, lv, net)

            k0, v0, k1, v1 = lax.fori_loop(
                0, NSTEPS, fb, (inf16, zero16i, inf16, zero16i))
            idxb[pl.ds(q * NS, L)] = v0
            idxb[pl.ds(q * NS + L, L)] = v1

        return carry

    lax.fori_loop(0, MCHUNK, qbody, 0)

    for dch in range(3):
        def gbody(g, c, dch=dch):
            iv = idxb[pl.ds(g * L, L)]
            vals = plsc.load_gather(pts, [iv + dch * N])
            ob[pl.ds(g * L, L)] = vals
            qc = plsc.load_gather(
                qv, [jnp.full((L,), (g // 2) * 3 + dch, jnp.int32)])
            ob2[pl.ds(g * L, L)] = vals - qc
            return c

        lax.fori_loop(0, NGRP, gbody, 0)
        pltpu.sync_copy(
            ob, gx.at[pl.ds(((b * 3 + dch) * M + mlo) * NS, MCHUNK * NS)])
        pltpu.sync_copy(
            ob2, nf.at[pl.ds(((b * CH + dch) * M + mlo) * NS, MCHUNK * NS)])

    NP = C // 2
    HB = 2 * N

    def _feat_src(pair):
        return features.at[pl.ds((b * C + 2 * pair) * N, HB)]

    OBH = MCHUNK * NS

    def _ob_dst(pair):
        off = ((b * CH + 3 + 2 * pair) * M + mlo) * NS
        return (nf.at[pl.ds(off, OBH)], nf.at[pl.ds(off + M * NS, OBH)])

    def _gather_pair(base, bufa, bufb, pair):
        def gb(g2, c):
            for uu in range(2):
                g = g2 * 2 + uu
                iv = idxb[pl.ds(g * L, L)]
                bufa[pl.ds(g * L, L)] = plsc.load_gather(pb, [iv + base])
                bufb[pl.ds(g * L, L)] = plsc.load_gather(pb, [iv + (base + N)])
            return c

        lax.fori_loop(0, NGRP // 2, gb, 0)
        d0, d1 = _ob_dst(pair)
        pltpu.async_copy(bufa, d0, sem2)
        pltpu.async_copy(bufb, d1, sem2)

    def _wait_out(bufa, bufb, pair):
        d0, d1 = _ob_dst(pair)
        pltpu.make_async_copy(bufa, d0, sem2).wait()
        pltpu.make_async_copy(bufb, d1, sem2).wait()

    pltpu.async_copy(_feat_src(0), pb.at[pl.ds(0, HB)], sem0)

    def fbody(t, carry):
        pa = 2 * t
        pltpu.make_async_copy(_feat_src(pa), pb.at[pl.ds(0, HB)], sem0).wait()
        pltpu.async_copy(_feat_src(pa + 1), pb.at[pl.ds(HB, HB)], sem1)

        @pl.when(t > 0)
        def _():
            _wait_out(ob, ob2, pa - 2)

        _gather_pair(0, ob, ob2, pa)
        pltpu.make_async_copy(_feat_src(pa + 1), pb.at[pl.ds(HB, HB)],
                              sem1).wait()
        nxt = jnp.minimum(pa + 2, NP - 1)
        pltpu.async_copy(_feat_src(nxt), pb.at[pl.ds(0, HB)], sem0)

        @pl.when(t > 0)
        def _():
            _wait_out(ob3, ob4, pa - 1)

        _gather_pair(HB, ob3, ob4, pa + 1)
        return carry

    lax.fori_loop(0, NP // 2, fbody, 0)
    pltpu.make_async_copy(_feat_src(NP - 1), pb.at[pl.ds(0, HB)], sem0).wait()
    _wait_out(ob, ob2, NP - 2)
    _wait_out(ob3, ob4, NP - 1)


def kernel(xyz, new_xyz, features):
    xyzT = jnp.transpose(xyz, (0, 2, 1)).reshape(-1)
    mesh = plsc.VectorSubcoreMesh(core_axis_name="c", subcore_axis_name="s",
                                  num_cores=2, num_subcores=16)
    nf, gx = pl.kernel(
        _body,
        out_type=[
            jax.ShapeDtypeStruct((B * CH * M * NS,), jnp.float32),
            jax.ShapeDtypeStruct((B * 3 * M * NS,), jnp.float32),
        ],
        mesh=mesh,
        compiler_params=pltpu.CompilerParams(needs_layout_passes=False),
        scratch_types=[
            pltpu.VMEM((3 * N,), jnp.float32),
            pltpu.VMEM((4 * N,), jnp.float32),
            pltpu.VMEM((N,), jnp.float32),
            pltpu.VMEM((N + L,), jnp.float32),
            pltpu.VMEM((MCHUNK * 3,), jnp.float32),
            pltpu.VMEM((WD * L,), jnp.int32),
            pltpu.VMEM((MCHUNK * NS,), jnp.int32),
            pltpu.VMEM((MCHUNK * NS,), jnp.float32),
            pltpu.VMEM((MCHUNK * NS,), jnp.float32),
            pltpu.VMEM((MCHUNK * NS,), jnp.float32),
            pltpu.VMEM((MCHUNK * NS,), jnp.float32),
            pltpu.SemaphoreType.DMA,
            pltpu.SemaphoreType.DMA,
            pltpu.SemaphoreType.DMA,
        ],
    )(xyzT, new_xyz.reshape(-1), features.reshape(-1))
    return (nf.reshape(B, CH, M, NS), gx.reshape(B, 3, M, NS))
